# Initial kernel scaffold; baseline (speedup 1.0000x reference)
import jax
import jax.numpy as jnp
from jax.experimental import pallas as pl


def kernel(x, levels):
    raise NotImplementedError("write your pallas kernel here")


# speedup vs baseline: 6.6182x; 6.6182x over previous
import functools

import jax
import jax.numpy as jnp
from jax import lax
from jax.experimental import pallas as pl
from jax.experimental.pallas import tpu as pltpu
from jax.experimental.pallas import tpu_sc as plsc

TOTAL = 16777216
NC, NS, L = 2, 16, 16
NW = NC * NS
PER_W = TOTAL // NW
C = 32768
NCH = PER_W // C
NV = C // L
NCELL = 256
LO, HI = -2.0, 3.0
SCALE = NCELL / (HI - LO)
NBUF = 3

_mesh = plsc.VectorSubcoreMesh(core_axis_name="c", subcore_axis_name="s")


@functools.partial(
    pl.kernel,
    mesh=_mesh,
    out_type=jax.ShapeDtypeStruct((TOTAL,), jnp.float32),
    scratch_types=(
        [pltpu.VMEM((NCELL,), jnp.float32)] * 2
        + [pltpu.VMEM((C,), jnp.float32)] * NBUF
        + [pltpu.SemaphoreType.DMA] * (2 * NBUF)
    ),
    compiler_params=pltpu.CompilerParams(needs_layout_passes=False),
)
def _quantize_sc(x_hbm, la_hbm, lb_hbm, out_hbm,
                 la_v, lb_v, b0, b1, b2, si0, si1, si2, so0, so1, so2):
    bufs = [b0, b1, b2]
    sin = [si0, si1, si2]
    sout = [so0, so1, so2]
    wid = lax.axis_index("s") * NC + lax.axis_index("c")
    base = wid * PER_W

    pltpu.sync_copy(la_hbm, la_v)
    pltpu.sync_copy(lb_hbm, lb_v)
    zeros = jnp.zeros((L,), jnp.int32)
    lo = plsc.load_gather(la_v, [zeros])
    hi = plsc.load_gather(lb_v, [zeros + (NCELL - 1)])

    def start_in(g, b):
        pltpu.make_async_copy(
            x_hbm.at[pl.ds(base + g * C, C)], bufs[b], sin[b]).start()

    def wait_in(b):
        pltpu.make_async_copy(
            x_hbm.at[pl.ds(base, C)], bufs[b], sin[b]).wait()

    def start_out(g, b):
        pltpu.make_async_copy(
            bufs[b], out_hbm.at[pl.ds(base + g * C, C)], sout[b]).start()

    def wait_out(b):
        pltpu.make_async_copy(
            bufs[b], out_hbm.at[pl.ds(base, C)], sout[b]).wait()

    def compute(b):
        buf = bufs[b]

        @plsc.parallel_loop(0, NV, step=1, unroll=8)
        def _vec(i):
            off = pl.multiple_of(i * L, L)
            xv = buf[pl.ds(off, L)]
            xc = jnp.minimum(jnp.maximum(xv, lo), hi)
            t = (xc - lo) * SCALE
            j = jnp.minimum(t.astype(jnp.int32), NCELL - 1)
            a = plsc.load_gather(la_v, [j])
            bb = plsc.load_gather(lb_v, [j])
            r = jnp.where((bb - xc) < (xc - a), bb, a)
            buf[pl.ds(off, L)] = (r - xc) + xc

    for g in range(min(NBUF, NCH)):
        start_in(g, g % NBUF)
    for g in range(NCH):
        b = g % NBUF
        wait_in(b)
        compute(b)
        start_out(g, b)
        if g >= 1 and g + 2 < NCH:
            wait_out((g - 1) % NBUF)
            start_in(g + 2, (g + 2) % NBUF)
    for g in range(max(0, NCH - 3), NCH):
        wait_out(g % NBUF)


def _build_luts(levels):
    w = (HI - LO) / NCELL
    cell_lefts = jnp.float32(LO) + jnp.arange(NCELL, dtype=jnp.float32) * jnp.float32(w)
    s = jnp.clip(jnp.searchsorted(levels, cell_lefts, side="right"), 1, levels.shape[0] - 1)
    return levels[s - 1], levels[s]


def kernel(x, levels):
    levels = levels.astype(jnp.float32)
    lut_a, lut_b = _build_luts(levels)
    return _quantize_sc(x, lut_a, lut_b)


# speedup vs baseline: 7.1694x; 1.0833x over previous
import functools

import jax
import jax.numpy as jnp
from jax import lax
from jax.experimental import pallas as pl
from jax.experimental.pallas import tpu as pltpu
from jax.experimental.pallas import tpu_sc as plsc

TOTAL = 16777216
NC, NS, L = 2, 16, 16
NW = NC * NS
PER_W = TOTAL // NW
C = 32768
NCH = PER_W // C
NV = C // L
NCELL = 256
LO, HI = -2.0, 3.0
SCALE = NCELL / (HI - LO)
NBUF = 3

_mesh = plsc.VectorSubcoreMesh(core_axis_name="c", subcore_axis_name="s")


@functools.partial(
    pl.kernel,
    mesh=_mesh,
    out_type=jax.ShapeDtypeStruct((TOTAL,), jnp.float32),
    scratch_types=(
        [pltpu.VMEM((NCELL + 1,), jnp.float32)] * 2
        + [pltpu.VMEM((C,), jnp.float32)] * NBUF
        + [pltpu.SemaphoreType.DMA] * (2 * NBUF)
    ),
    compiler_params=pltpu.CompilerParams(needs_layout_passes=False),
)
def _quantize_sc(x_hbm, la_hbm, lb_hbm, out_hbm,
                 la_v, lb_v, b0, b1, b2, si0, si1, si2, so0, so1, so2):
    bufs = [b0, b1, b2]
    sin = [si0, si1, si2]
    sout = [so0, so1, so2]
    wid = lax.axis_index("s") * NC + lax.axis_index("c")
    base = wid * PER_W

    pltpu.sync_copy(la_hbm, la_v)
    pltpu.sync_copy(lb_hbm, lb_v)
    zeros = jnp.zeros((L,), jnp.int32)
    lo = plsc.load_gather(la_v, [zeros])
    hi = plsc.load_gather(lb_v, [zeros + NCELL])

    def start_in(g, b):
        pltpu.make_async_copy(
            x_hbm.at[pl.ds(base + g * C, C)], bufs[b], sin[b]).start()

    def wait_in(b):
        pltpu.make_async_copy(
            x_hbm.at[pl.ds(base, C)], bufs[b], sin[b]).wait()

    def start_out(g, b):
        pltpu.make_async_copy(
            bufs[b], out_hbm.at[pl.ds(base + g * C, C)], sout[b]).start()

    def wait_out(b):
        pltpu.make_async_copy(
            bufs[b], out_hbm.at[pl.ds(base, C)], sout[b]).wait()

    def compute(b):
        buf = bufs[b]

        @plsc.parallel_loop(0, NV, step=1, unroll=16)
        def _vec(i):
            off = pl.multiple_of(i * L, L)
            xv = buf[pl.ds(off, L)]
            xc = jnp.minimum(jnp.maximum(xv, lo), hi)
            j = ((xc - lo) * SCALE).astype(jnp.int32)
            a = plsc.load_gather(la_v, [j])
            bb = plsc.load_gather(lb_v, [j])
            buf[pl.ds(off, L)] = jnp.where((bb - xc) < (xc - a), bb, a)

    for g in range(NBUF):
        start_in(g, g)
    wait_in(0)
    compute(0)
    start_out(0, 0)

    @pl.loop(1, NCH - NBUF, step=NBUF)
    def _chunks(gv):
        for k in range(NBUF):
            g = gv + k
            b = (1 + k) % NBUF
            wait_in(b)
            compute(b)
            start_out(g, b)
            wait_out((b + 2) % NBUF)
            start_in(g + 2, (b + 2) % NBUF)

    for g in range(NCH - NBUF, NCH):
        b = g % NBUF
        wait_in(b)
        compute(b)
        start_out(g, b)
        wait_out((b + 2) % NBUF)
        if g + 2 < NCH:
            start_in(g + 2, (b + 2) % NBUF)
    wait_out((NCH - 1) % NBUF)


def _build_luts(levels):
    w = (HI - LO) / NCELL
    cell_lefts = jnp.float32(LO) + jnp.arange(NCELL + 1, dtype=jnp.float32) * jnp.float32(w)
    s = jnp.clip(jnp.searchsorted(levels, cell_lefts, side="right"), 1, levels.shape[0] - 1)
    return levels[s - 1], levels[s]


def kernel(x, levels):
    levels = levels.astype(jnp.float32)
    lut_a, lut_b = _build_luts(levels)
    return _quantize_sc(x, lut_a, lut_b)


# speedup vs baseline: 10.3198x; 1.4394x over previous
import functools

import jax
import jax.numpy as jnp
from jax import lax
from jax.experimental import pallas as pl
from jax.experimental.pallas import tpu as pltpu
from jax.experimental.pallas import tpu_sc as plsc

TOTAL = 16777216
NC, NS, L = 2, 16, 16
NW = NC * NS
PER_W = TOTAL // NW
C = 32768
NCH = PER_W // C
NV = C // L
NCELL = 256
LO, HI = -2.0, 3.0
SCALE = NCELL / (HI - LO)
NBUF = 3

_mesh = plsc.VectorSubcoreMesh(core_axis_name="c", subcore_axis_name="s")


@functools.partial(
    pl.kernel,
    mesh=_mesh,
    out_type=jax.ShapeDtypeStruct((TOTAL,), jnp.float32),
    scratch_types=(
        [pltpu.VMEM((NCELL + 1,), jnp.float32)] * 2
        + [pltpu.VMEM((C,), jnp.float32)] * NBUF
        + [pltpu.SemaphoreType.DMA] * (2 * NBUF)
    ),
    compiler_params=pltpu.CompilerParams(needs_layout_passes=False),
)
def _quantize_sc(x_hbm, la_hbm, lb_hbm, out_hbm,
                 la_v, lb_v, b0, b1, b2, si0, si1, si2, so0, so1, so2):
    bufs = [b0, b1, b2]
    sin = [si0, si1, si2]
    sout = [so0, so1, so2]
    wid = lax.axis_index("s") * NC + lax.axis_index("c")
    base = wid * PER_W

    pltpu.sync_copy(la_hbm, la_v)
    pltpu.sync_copy(lb_hbm, lb_v)
    zeros = jnp.zeros((L,), jnp.int32)
    lo = plsc.load_gather(la_v, [zeros])
    hi = plsc.load_gather(lb_v, [zeros + NCELL])

    def start_in(g, b):
        pltpu.make_async_copy(
            x_hbm.at[pl.ds(base + g * C, C)], bufs[b], sin[b]).start()

    def wait_in(b):
        pltpu.make_async_copy(
            x_hbm.at[pl.ds(base, C)], bufs[b], sin[b]).wait()

    def start_out(g, b):
        pltpu.make_async_copy(
            bufs[b], out_hbm.at[pl.ds(base + g * C, C)], sout[b]).start()

    def wait_out(b):
        pltpu.make_async_copy(
            bufs[b], out_hbm.at[pl.ds(base, C)], sout[b]).wait()

    def compute(b):
        buf = bufs[b]

        @plsc.parallel_loop(0, NV, step=1, unroll=16)
        def _vec(i):
            off = pl.multiple_of(i * L, L)
            xv = buf[pl.ds(off, L)]
            xc = jnp.minimum(jnp.maximum(xv, lo), hi)
            j = ((xc - lo) * SCALE).astype(jnp.int32)
            a = plsc.load_gather(la_v, [j])
            bb = plsc.load_gather(lb_v, [j])
            buf[pl.ds(off, L)] = jnp.where((bb - xc) < (xc - a), bb, a)

    for g in range(NBUF):
        start_in(g, g)
    wait_in(0)
    compute(0)
    start_out(0, 0)

    @pl.loop(1, NCH - NBUF, step=NBUF)
    def _chunks(gv):
        for k in range(NBUF):
            g = gv + k
            b = (1 + k) % NBUF
            wait_in(b)
            compute(b)
            start_out(g, b)
            wait_out((b + 2) % NBUF)
            start_in(g + 2, (b + 2) % NBUF)

    for g in range(NCH - NBUF, NCH):
        b = g % NBUF
        wait_in(b)
        compute(b)
        start_out(g, b)
        wait_out((b + 2) % NBUF)
        if g + 2 < NCH:
            start_in(g + 2, (b + 2) % NBUF)
    wait_out((NCH - 1) % NBUF)


def _build_luts(levels):
    w = (HI - LO) / NCELL
    cell_lefts = jnp.float32(LO) + jnp.arange(NCELL + 1, dtype=jnp.float32) * jnp.float32(w)
    s = jnp.sum(levels[None, :] <= cell_lefts[:, None], axis=1, dtype=jnp.int32)
    s = jnp.clip(s, 1, levels.shape[0] - 1)
    return jnp.take(levels, s - 1), jnp.take(levels, s)


def kernel(x, levels):
    levels = levels.astype(jnp.float32)
    lut_a, lut_b = _build_luts(levels)
    return _quantize_sc(x, lut_a, lut_b)
